# baseline (speedup 1.0000x reference)
import functools
import jax
import jax.numpy as jnp
from jax import lax
from jax.experimental import pallas as pl
from jax.experimental.pallas import tpu as pltpu
from jax.experimental.pallas import tpu_sc as plsc

DIM = 64
NUM_CTX = 5
NC = 2
NS = 16
NW = NC * NS
CB = 128
NLV = DIM // 16


def _body(tgt_i, ctx_i, tgt_tab, ctx_tab, out, tgt_idx_v, ctx_idx_v,
          tgt_rows, ctx_rows, out_v, sem, *, nchunk):
    ppc = CB * NUM_CTX
    w = lax.axis_index("s") * NC + lax.axis_index("c")
    lanes = lax.iota(jnp.int32, 16)
    pltpu.sync_copy(tgt_i.at[w], tgt_idx_v)
    pltpu.sync_copy(ctx_i.at[w], ctx_idx_v)

    for k in range(nchunk):
        pltpu.async_copy(tgt_tab.at[tgt_idx_v.at[k]], tgt_rows, sem).wait()
        for c in range(NUM_CTX):
            pltpu.async_copy(ctx_tab.at[ctx_idx_v.at[k, c]],
                             ctx_rows.at[pl.ds(c * CB, CB)], sem).wait()

        def gstep(g, carry, k=k):
            b_iota = g * 16 + lanes
            pvec = [b_iota * NUM_CTX + s for s in range(NUM_CTX)]
            accs = [jnp.zeros((16,), jnp.float32) for _ in range(NUM_CTX)]
            for e in range(DIM):
                e_splat = jnp.full((16,), e, jnp.int32)
                wcol = plsc.load_gather(tgt_rows, [b_iota, e_splat])
                for s in range(NUM_CTX):
                    xcol = plsc.load_gather(ctx_rows, [pvec[s], e_splat])
                    accs[s] = accs[s] + wcol * xcol
            ks = jnp.full((16,), k, jnp.int32)
            for s in range(NUM_CTX):
                plsc.store_scatter(out_v, [ks, pvec[s]], accs[s])
            return carry

        lax.fori_loop(0, CB // 16, gstep, 0)

    pltpu.sync_copy(out_v, out.at[w])


def kernel(target, context, target_table, context_table):
    batch, num_ctx = context.shape
    assert num_ctx == NUM_CTX and batch % (NW * CB) == 0
    nchunk = batch // (NW * CB)
    ppc = CB * NUM_CTX

    tgt_i = target.astype(jnp.int32).reshape(NW, nchunk, CB)
    ctx_i = context.astype(jnp.int32).reshape(NW, nchunk, NUM_CTX, CB)

    mesh = plsc.VectorSubcoreMesh(core_axis_name="c", subcore_axis_name="s")
    grid_kernel = pl.kernel(
        functools.partial(_body, nchunk=nchunk),
        out_type=jax.ShapeDtypeStruct((NW, nchunk, ppc), jnp.float32),
        mesh=mesh,
        scratch_types=[
            pltpu.VMEM((nchunk, CB), jnp.int32),
            pltpu.VMEM((nchunk, NUM_CTX, CB), jnp.int32),
            pltpu.VMEM((CB, DIM), jnp.float32),
            pltpu.VMEM((ppc, DIM), jnp.float32),
            pltpu.VMEM((nchunk, ppc), jnp.float32),
            pltpu.SemaphoreType.DMA,
        ],
        compiler_params=pltpu.CompilerParams(
            needs_layout_passes=False, use_tc_tiling_on_sc=False),
    )
    out = grid_kernel(tgt_i, ctx_i, target_table, context_table)
    return out.reshape(batch, NUM_CTX)
